# Initial kernel scaffold; baseline (speedup 1.0000x reference)
import jax
import jax.numpy as jnp
from jax.experimental import pallas as pl


def kernel(x, edge_index, edge_embed, W_lin1, W_radial1, W_radial2, W_lin2):
    raise NotImplementedError("write your pallas kernel here")


# speedup vs baseline: 2.7447x; 2.7447x over previous
import functools
import math

import jax
import jax.numpy as jnp
from jax import lax
from jax.experimental import pallas as pl
from jax.experimental.pallas import tpu as pltpu
from jax.experimental.pallas import tpu_sc as plsc

N = 10000
E = 320000
D = 128
NB = 16
H = 64
INV_SQRT_DEG = 1.0 / math.sqrt(32.0)
LOG2 = math.log(2.0)

NC = 2
NS = 16
NW = NC * NS
CHUNK = 128
NCHUNKS = E // CHUNK
MAXG = -(-NCHUNKS // NW)
ROWS_PER_TILE = 624
ROWS_TAIL = N - NS * ROWS_PER_TILE


def _ssp(v):
    return jax.nn.softplus(v) - LOG2


def _lin1_body(x_ref, w_ref, o_ref):
    o_ref[...] = jnp.dot(x_ref[...], w_ref[...], preferred_element_type=jnp.float32)


def _radial_body(ee_ref, w1_ref, w2_ref, o_ref):
    mid = _ssp(jnp.dot(ee_ref[...], w1_ref[...], preferred_element_type=jnp.float32))
    o_ref[...] = jnp.dot(mid, w2_ref[...], preferred_element_type=jnp.float32)


def _final_body(parts_ref, x_ref, w_ref, o_ref):
    p = parts_ref[...]
    agg = (p[0] + p[1]) * INV_SQRT_DEG
    o_ref[...] = _ssp(jnp.dot(agg, w_ref[...], preferred_element_type=jnp.float32)) + x_ref[...]


def _lin1(x, w):
    blk = 1000
    return pl.pallas_call(
        _lin1_body,
        grid=(N // blk,),
        in_specs=[pl.BlockSpec((blk, D), lambda i: (i, 0)),
                  pl.BlockSpec((D, D), lambda i: (0, 0))],
        out_specs=pl.BlockSpec((blk, D), lambda i: (i, 0)),
        out_shape=jax.ShapeDtypeStruct((N, D), jnp.float32),
    )(x, w)


def _radial(ee, w1, w2):
    blk = 4000
    return pl.pallas_call(
        _radial_body,
        grid=(E // blk,),
        in_specs=[pl.BlockSpec((blk, NB), lambda i: (i, 0)),
                  pl.BlockSpec((NB, H), lambda i: (0, 0)),
                  pl.BlockSpec((H, D), lambda i: (0, 0))],
        out_specs=pl.BlockSpec((blk, D), lambda i: (i, 0)),
        out_shape=jax.ShapeDtypeStruct((E, D), jnp.float32),
    )(ee, w1, w2)


def _final(parts, x, w):
    blk = 1000
    return pl.pallas_call(
        _final_body,
        grid=(N // blk,),
        in_specs=[pl.BlockSpec((NC, blk, D), lambda i: (0, i, 0)),
                  pl.BlockSpec((blk, D), lambda i: (i, 0)),
                  pl.BlockSpec((D, D), lambda i: (0, 0))],
        out_specs=pl.BlockSpec((blk, D), lambda i: (i, 0)),
        out_shape=jax.ShapeDtypeStruct((N, D), jnp.float32),
    )(parts, x, w)


def _sc_body(h_hbm, src_hbm, dst_hbm, radial_hbm, out_hbm,
             src_v, dst_v, rows_v, rad_v, acc_sh, sem):
    cid = lax.axis_index("c")
    sid = lax.axis_index("s")
    wid = sid * NC + cid

    def zero_body(c, carry):
        for dd in range(D // 16):
            rows_v[c, pl.ds(dd * 16, 16)] = jnp.zeros((16,), jnp.float32)
        return carry
    lax.fori_loop(0, CHUNK, zero_body, 0)

    row0 = sid * ROWS_PER_TILE
    done = 0
    while done < ROWS_PER_TILE:
        n = min(CHUNK, ROWS_PER_TILE - done)
        pltpu.sync_copy(rows_v.at[pl.ds(0, n)], acc_sh.at[pl.ds(row0 + done, n)])
        done += n

    @pl.when(sid == NS - 1)
    def _():
        pltpu.sync_copy(rows_v.at[pl.ds(0, ROWS_TAIL)],
                        acc_sh.at[pl.ds(NS * ROWS_PER_TILE, ROWS_TAIL)])
    plsc.subcore_barrier()

    def chunk_body(g, carry):
        j = wid + g * NW

        @pl.when(j < NCHUNKS)
        def _():
            base = j * CHUNK
            pltpu.sync_copy(src_hbm.at[pl.ds(base, CHUNK)], src_v)
            pltpu.sync_copy(dst_hbm.at[pl.ds(base, CHUNK)], dst_v)
            pltpu.async_copy(h_hbm.at[src_v], rows_v, sem).wait()
            pltpu.sync_copy(radial_hbm.at[pl.ds(base, CHUNK)], rad_v)

            def mul_body(c, carry2):
                for dd in range(D // 16):
                    sl = pl.ds(dd * 16, 16)
                    rows_v[c, sl] = rows_v[c, sl] * rad_v[c, sl]
                return carry2
            lax.fori_loop(0, CHUNK, mul_body, 0)

            pltpu.sync_copy(rows_v, acc_sh.at[dst_v], add=True)
        return carry
    lax.fori_loop(0, MAXG, chunk_body, 0)

    plsc.subcore_barrier()

    pltpu.sync_copy(acc_sh.at[pl.ds(row0, ROWS_PER_TILE)],
                    out_hbm.at[cid, pl.ds(row0, ROWS_PER_TILE)])

    @pl.when(sid == NS - 1)
    def _():
        pltpu.sync_copy(acc_sh.at[pl.ds(NS * ROWS_PER_TILE, ROWS_TAIL)],
                        out_hbm.at[cid, pl.ds(NS * ROWS_PER_TILE, ROWS_TAIL)])


def _sc_scatter(h, src, dst, radial):
    mesh = plsc.VectorSubcoreMesh(core_axis_name="c", subcore_axis_name="s")
    f = pl.kernel(
        _sc_body, mesh=mesh,
        out_type=jax.ShapeDtypeStruct((NC, N, D), jnp.float32),
        scratch_types=[
            pltpu.VMEM((CHUNK,), jnp.int32),
            pltpu.VMEM((CHUNK,), jnp.int32),
            pltpu.VMEM((CHUNK, D), jnp.float32),
            pltpu.VMEM((CHUNK, D), jnp.float32),
            pltpu.VMEM_SHARED((N, D), jnp.float32),
            pltpu.SemaphoreType.DMA,
        ],
    )
    return f(h, src, dst, radial)


def kernel(x, edge_index, edge_embed, W_lin1, W_radial1, W_radial2, W_lin2):
    src = edge_index[0]
    dst = edge_index[1]
    h = _lin1(x, W_lin1)
    radial = _radial(edge_embed, W_radial1, W_radial2)
    parts = _sc_scatter(h, src, dst, radial)
    return _final(parts, x, W_lin2)


# speedup vs baseline: 3.7160x; 1.3539x over previous
import functools
import math

import jax
import jax.numpy as jnp
from jax import lax
from jax.experimental import pallas as pl
from jax.experimental.pallas import tpu as pltpu
from jax.experimental.pallas import tpu_sc as plsc

N = 10000
E = 320000
D = 128
NB = 16
H = 64
INV_SQRT_DEG = 1.0 / math.sqrt(32.0)
LOG2 = math.log(2.0)

NC = 2
NS = 16
NW = NC * NS
CHUNK = 80
NCH = 125
EDGES_PER_W = NCH * CHUNK
ROWS_PER_TILE = 624
ROWS_TAIL = N - NS * ROWS_PER_TILE


def _ssp(v):
    return jax.nn.softplus(v) - LOG2


def _lin1_body(x_ref, w_ref, o_ref):
    o_ref[...] = jnp.dot(x_ref[...], w_ref[...], preferred_element_type=jnp.float32)


def _radial_body(ee_ref, w1_ref, w2_ref, o_ref):
    mid = _ssp(jnp.dot(ee_ref[...], w1_ref[...], preferred_element_type=jnp.float32))
    o_ref[...] = jnp.dot(mid, w2_ref[...], preferred_element_type=jnp.float32)


def _final_body(parts_ref, x_ref, w_ref, o_ref):
    p = parts_ref[...]
    agg = (p[0] + p[1]) * INV_SQRT_DEG
    o_ref[...] = _ssp(jnp.dot(agg, w_ref[...], preferred_element_type=jnp.float32)) + x_ref[...]


def _lin1(x, w):
    blk = 1000
    return pl.pallas_call(
        _lin1_body,
        grid=(N // blk,),
        in_specs=[pl.BlockSpec((blk, D), lambda i: (i, 0)),
                  pl.BlockSpec((D, D), lambda i: (0, 0))],
        out_specs=pl.BlockSpec((blk, D), lambda i: (i, 0)),
        out_shape=jax.ShapeDtypeStruct((N, D), jnp.float32),
    )(x, w)


def _radial(ee, w1, w2):
    blk = 4000
    return pl.pallas_call(
        _radial_body,
        grid=(E // blk,),
        in_specs=[pl.BlockSpec((blk, NB), lambda i: (i, 0)),
                  pl.BlockSpec((NB, H), lambda i: (0, 0)),
                  pl.BlockSpec((H, D), lambda i: (0, 0))],
        out_specs=pl.BlockSpec((blk, D), lambda i: (i, 0)),
        out_shape=jax.ShapeDtypeStruct((E, D), jnp.float32),
    )(ee, w1, w2)


def _final(parts, x, w):
    blk = 1000
    return pl.pallas_call(
        _final_body,
        grid=(N // blk,),
        in_specs=[pl.BlockSpec((NC, blk, D), lambda i: (0, i, 0)),
                  pl.BlockSpec((blk, D), lambda i: (i, 0)),
                  pl.BlockSpec((D, D), lambda i: (0, 0))],
        out_specs=pl.BlockSpec((blk, D), lambda i: (i, 0)),
        out_shape=jax.ShapeDtypeStruct((N, D), jnp.float32),
    )(parts, x, w)


def _sc_body(h_hbm, src_hbm, dst_hbm, radial_hbm, out_hbm,
             src0, src1, dst0, dst1, rows0, rows1, rad0, rad1, acc_sh,
             dsem0, dsem1, ssem0, ssem1):
    cid = lax.axis_index("c")
    sid = lax.axis_index("s")
    wid = sid * NC + cid
    ebase = wid * EDGES_PER_W
    bufs = ((src0, dst0, rows0, rad0, dsem0, ssem0),
            (src1, dst1, rows1, rad1, dsem1, ssem1))

    def zero_body(c, carry):
        for dd in range(D // 16):
            rows0[c, pl.ds(dd * 16, 16)] = jnp.zeros((16,), jnp.float32)
        return carry
    lax.fori_loop(0, CHUNK, zero_body, 0)

    row0 = sid * ROWS_PER_TILE
    done = 0
    while done < ROWS_PER_TILE:
        n = min(CHUNK, ROWS_PER_TILE - done)
        pltpu.sync_copy(rows0.at[pl.ds(0, n)], acc_sh.at[pl.ds(row0 + done, n)])
        done += n

    @pl.when(sid == NS - 1)
    def _():
        pltpu.sync_copy(rows0.at[pl.ds(0, ROWS_TAIL)],
                        acc_sh.at[pl.ds(NS * ROWS_PER_TILE, ROWS_TAIL)])
    plsc.subcore_barrier()

    def fetch(B, base):
        srcv, dstv, rows, rad, dsem, _ = bufs[B]
        pltpu.async_copy(src_hbm.at[pl.ds(base, CHUNK)], srcv, dsem)
        pltpu.async_copy(dst_hbm.at[pl.ds(base, CHUNK)], dstv, dsem)
        pltpu.async_copy(radial_hbm.at[pl.ds(base, CHUNK)], rad, dsem)

    def wait_fetch_idx(B):
        srcv, dstv, rows, rad, dsem, _ = bufs[B]
        pltpu.make_async_copy(src_hbm.at[pl.ds(0, CHUNK)], srcv, dsem).wait()
        pltpu.make_async_copy(dst_hbm.at[pl.ds(0, CHUNK)], dstv, dsem).wait()
        pltpu.make_async_copy(radial_hbm.at[pl.ds(0, CHUNK)], rad, dsem).wait()

    def gather(B):
        srcv, _, rows, _, dsem, _ = bufs[B]
        pltpu.async_copy(h_hbm.at[srcv], rows, dsem)

    def wait_gather(B):
        srcv, _, rows, _, dsem, _ = bufs[B]
        pltpu.make_async_copy(h_hbm.at[srcv], rows, dsem).wait()

    def mult(B):
        _, _, rows, rad, _, _ = bufs[B]

        def mul_body(c, carry2):
            for dd in range(D // 16):
                sl = pl.ds(dd * 16, 16)
                rows[c, sl] = rows[c, sl] * rad[c, sl]
            return carry2
        lax.fori_loop(0, CHUNK, mul_body, 0)

    def scat(B):
        _, dstv, rows, _, _, ssem = bufs[B]
        pltpu.async_copy(rows, acc_sh.at[dstv], ssem, add=True)

    def wait_scat(B):
        _, dstv, rows, _, _, ssem = bufs[B]
        pltpu.make_async_copy(rows, acc_sh.at[dstv], ssem).wait()

    fetch(0, ebase)
    wait_fetch_idx(0)
    gather(0)

    def pair_body(gg, carry):
        j0 = 2 * gg

        @pl.when(gg > 0)
        def _():
            wait_scat(1)
        fetch(1, ebase + (j0 + 1) * CHUNK)
        wait_gather(0)
        mult(0)
        scat(0)
        wait_fetch_idx(1)
        gather(1)

        @pl.when(gg < NCH // 2 - 1)
        def _():
            wait_scat(0)
            fetch(0, ebase + (j0 + 2) * CHUNK)
        wait_gather(1)
        mult(1)
        scat(1)

        @pl.when(gg < NCH // 2 - 1)
        def _():
            wait_fetch_idx(0)
            gather(0)
        return carry
    lax.fori_loop(0, NCH // 2, pair_body, 0)

    wait_scat(0)
    fetch(0, ebase + (NCH - 1) * CHUNK)
    wait_fetch_idx(0)
    gather(0)
    wait_gather(0)
    mult(0)
    scat(0)
    wait_scat(0)
    wait_scat(1)

    plsc.subcore_barrier()

    pltpu.sync_copy(acc_sh.at[pl.ds(row0, ROWS_PER_TILE)],
                    out_hbm.at[cid, pl.ds(row0, ROWS_PER_TILE)])

    @pl.when(sid == NS - 1)
    def _():
        pltpu.sync_copy(acc_sh.at[pl.ds(NS * ROWS_PER_TILE, ROWS_TAIL)],
                        out_hbm.at[cid, pl.ds(NS * ROWS_PER_TILE, ROWS_TAIL)])


def _sc_scatter(h, src, dst, radial):
    mesh = plsc.VectorSubcoreMesh(core_axis_name="c", subcore_axis_name="s")
    f = pl.kernel(
        _sc_body, mesh=mesh,
        out_type=jax.ShapeDtypeStruct((NC, N, D), jnp.float32),
        scratch_types=[
            pltpu.VMEM((CHUNK,), jnp.int32),
            pltpu.VMEM((CHUNK,), jnp.int32),
            pltpu.VMEM((CHUNK,), jnp.int32),
            pltpu.VMEM((CHUNK,), jnp.int32),
            pltpu.VMEM((CHUNK, D), jnp.float32),
            pltpu.VMEM((CHUNK, D), jnp.float32),
            pltpu.VMEM((CHUNK, D), jnp.float32),
            pltpu.VMEM((CHUNK, D), jnp.float32),
            pltpu.VMEM_SHARED((N, D), jnp.float32),
            pltpu.SemaphoreType.DMA,
            pltpu.SemaphoreType.DMA,
            pltpu.SemaphoreType.DMA,
            pltpu.SemaphoreType.DMA,
        ],
    )
    return f(h, src, dst, radial)


def kernel(x, edge_index, edge_embed, W_lin1, W_radial1, W_radial2, W_lin2):
    src = edge_index[0]
    dst = edge_index[1]
    h = _lin1(x, W_lin1)
    radial = _radial(edge_embed, W_radial1, W_radial2)
    parts = _sc_scatter(h, src, dst, radial)
    return _final(parts, x, W_lin2)


# speedup vs baseline: 4.6165x; 1.2423x over previous
import functools
import math

import jax
import jax.numpy as jnp
from jax import lax
from jax.experimental import pallas as pl
from jax.experimental.pallas import tpu as pltpu
from jax.experimental.pallas import tpu_sc as plsc

N = 10000
E = 320000
D = 128
NB = 16
H = 64
INV_SQRT_DEG = 1.0 / math.sqrt(32.0)
LOG2 = math.log(2.0)

NC = 2
NS = 16
NW = NC * NS
CHUNK = 80
NCH = 125
EDGES_PER_W = NCH * CHUNK
ROWS_PER_TILE = 624
ROWS_TAIL = N - NS * ROWS_PER_TILE


def _ssp(v):
    return jax.nn.softplus(v) - LOG2


def _lin1_body(x_ref, w_ref, o_ref):
    o_ref[...] = jnp.dot(x_ref[...], w_ref[...], preferred_element_type=jnp.float32)


def _radial_body(eet_ref, w1_ref, w2_ref, o_ref):
    mid = _ssp(lax.dot_general(eet_ref[...], w1_ref[...],
                               (((0,), (0,)), ((), ())),
                               preferred_element_type=jnp.float32))
    o_ref[...] = jnp.dot(mid, w2_ref[...], preferred_element_type=jnp.float32)


def _final_body(parts_ref, x_ref, w_ref, o_ref):
    p = parts_ref[...]
    agg = (p[0] + p[1]) * INV_SQRT_DEG
    o_ref[...] = _ssp(jnp.dot(agg, w_ref[...], preferred_element_type=jnp.float32)) + x_ref[...]


def _lin1(x, w):
    blk = 1000
    return pl.pallas_call(
        _lin1_body,
        grid=(N // blk,),
        in_specs=[pl.BlockSpec((blk, D), lambda i: (i, 0)),
                  pl.BlockSpec((D, D), lambda i: (0, 0))],
        out_specs=pl.BlockSpec((blk, D), lambda i: (i, 0)),
        out_shape=jax.ShapeDtypeStruct((N, D), jnp.float32),
    )(x, w)


def _radial(eet, w1, w2):
    blk = 3200
    return pl.pallas_call(
        _radial_body,
        grid=(E // blk,),
        in_specs=[pl.BlockSpec((NB, blk), lambda i: (0, i)),
                  pl.BlockSpec((NB, H), lambda i: (0, 0)),
                  pl.BlockSpec((H, D), lambda i: (0, 0))],
        out_specs=pl.BlockSpec((blk, D), lambda i: (i, 0)),
        out_shape=jax.ShapeDtypeStruct((E, D), jnp.float32),
    )(eet, w1, w2)


def _final(parts, x, w):
    blk = 1000
    return pl.pallas_call(
        _final_body,
        grid=(N // blk,),
        in_specs=[pl.BlockSpec((NC, blk, D), lambda i: (0, i, 0)),
                  pl.BlockSpec((blk, D), lambda i: (i, 0)),
                  pl.BlockSpec((D, D), lambda i: (0, 0))],
        out_specs=pl.BlockSpec((blk, D), lambda i: (i, 0)),
        out_shape=jax.ShapeDtypeStruct((N, D), jnp.float32),
    )(parts, x, w)


def _sc_body(h_hbm, src_hbm, dst_hbm, radial_hbm, out_hbm,
             src0, src1, dst0, dst1, rows0, rows1, rad0, rad1, acc_sh,
             dsem0, dsem1, ssem0, ssem1):
    cid = lax.axis_index("c")
    sid = lax.axis_index("s")
    wid = sid * NC + cid
    ebase = wid * EDGES_PER_W
    bufs = ((src0, dst0, rows0, rad0, dsem0, ssem0),
            (src1, dst1, rows1, rad1, dsem1, ssem1))

    def zero_body(c, carry):
        for dd in range(D // 16):
            rows0[c, pl.ds(dd * 16, 16)] = jnp.zeros((16,), jnp.float32)
        return carry
    lax.fori_loop(0, CHUNK, zero_body, 0)

    row0 = sid * ROWS_PER_TILE
    done = 0
    while done < ROWS_PER_TILE:
        n = min(CHUNK, ROWS_PER_TILE - done)
        pltpu.sync_copy(rows0.at[pl.ds(0, n)], acc_sh.at[pl.ds(row0 + done, n)])
        done += n

    @pl.when(sid == NS - 1)
    def _():
        pltpu.sync_copy(rows0.at[pl.ds(0, ROWS_TAIL)],
                        acc_sh.at[pl.ds(NS * ROWS_PER_TILE, ROWS_TAIL)])
    plsc.subcore_barrier()

    def fetch(B, base):
        srcv, dstv, rows, rad, dsem, _ = bufs[B]
        pltpu.async_copy(src_hbm.at[pl.ds(base, CHUNK)], srcv, dsem)
        pltpu.async_copy(dst_hbm.at[pl.ds(base, CHUNK)], dstv, dsem)
        pltpu.async_copy(radial_hbm.at[pl.ds(base, CHUNK)], rad, dsem)

    def wait_fetch_idx(B):
        srcv, dstv, rows, rad, dsem, _ = bufs[B]
        pltpu.make_async_copy(src_hbm.at[pl.ds(0, CHUNK)], srcv, dsem).wait()
        pltpu.make_async_copy(dst_hbm.at[pl.ds(0, CHUNK)], dstv, dsem).wait()
        pltpu.make_async_copy(radial_hbm.at[pl.ds(0, CHUNK)], rad, dsem).wait()

    def gather(B):
        srcv, _, rows, _, dsem, _ = bufs[B]
        pltpu.async_copy(h_hbm.at[srcv], rows, dsem)

    def wait_gather(B):
        srcv, _, rows, _, dsem, _ = bufs[B]
        pltpu.make_async_copy(h_hbm.at[srcv], rows, dsem).wait()

    def mult(B):
        _, _, rows, rad, _, _ = bufs[B]

        def mul_body(c, carry2):
            for dd in range(D // 16):
                sl = pl.ds(dd * 16, 16)
                rows[c, sl] = rows[c, sl] * rad[c, sl]
            return carry2
        lax.fori_loop(0, CHUNK, mul_body, 0)

    def scat(B):
        _, dstv, rows, _, _, ssem = bufs[B]
        pltpu.async_copy(rows, acc_sh.at[dstv], ssem, add=True)

    def wait_scat(B):
        _, dstv, rows, _, _, ssem = bufs[B]
        pltpu.make_async_copy(rows, acc_sh.at[dstv], ssem).wait()

    fetch(0, ebase)
    wait_fetch_idx(0)
    gather(0)

    def pair_body(gg, carry):
        j0 = 2 * gg

        @pl.when(gg > 0)
        def _():
            wait_scat(1)
        fetch(1, ebase + (j0 + 1) * CHUNK)
        wait_gather(0)
        mult(0)
        scat(0)
        wait_fetch_idx(1)
        gather(1)

        @pl.when(gg < NCH // 2 - 1)
        def _():
            wait_scat(0)
            fetch(0, ebase + (j0 + 2) * CHUNK)
        wait_gather(1)
        mult(1)
        scat(1)

        @pl.when(gg < NCH // 2 - 1)
        def _():
            wait_fetch_idx(0)
            gather(0)
        return carry
    lax.fori_loop(0, NCH // 2, pair_body, 0)

    wait_scat(0)
    fetch(0, ebase + (NCH - 1) * CHUNK)
    wait_fetch_idx(0)
    gather(0)
    wait_gather(0)
    mult(0)
    scat(0)
    wait_scat(0)
    wait_scat(1)

    plsc.subcore_barrier()

    pltpu.sync_copy(acc_sh.at[pl.ds(row0, ROWS_PER_TILE)],
                    out_hbm.at[cid, pl.ds(row0, ROWS_PER_TILE)])

    @pl.when(sid == NS - 1)
    def _():
        pltpu.sync_copy(acc_sh.at[pl.ds(NS * ROWS_PER_TILE, ROWS_TAIL)],
                        out_hbm.at[cid, pl.ds(NS * ROWS_PER_TILE, ROWS_TAIL)])


def _sc_scatter(h, src, dst, radial):
    mesh = plsc.VectorSubcoreMesh(core_axis_name="c", subcore_axis_name="s")
    f = pl.kernel(
        _sc_body, mesh=mesh,
        out_type=jax.ShapeDtypeStruct((NC, N, D), jnp.float32),
        scratch_types=[
            pltpu.VMEM((CHUNK,), jnp.int32),
            pltpu.VMEM((CHUNK,), jnp.int32),
            pltpu.VMEM((CHUNK,), jnp.int32),
            pltpu.VMEM((CHUNK,), jnp.int32),
            pltpu.VMEM((CHUNK, D), jnp.float32),
            pltpu.VMEM((CHUNK, D), jnp.float32),
            pltpu.VMEM((CHUNK, D), jnp.float32),
            pltpu.VMEM((CHUNK, D), jnp.float32),
            pltpu.VMEM_SHARED((N, D), jnp.float32),
            pltpu.SemaphoreType.DMA,
            pltpu.SemaphoreType.DMA,
            pltpu.SemaphoreType.DMA,
            pltpu.SemaphoreType.DMA,
        ],
    )
    return f(h, src, dst, radial)


def kernel(x, edge_index, edge_embed, W_lin1, W_radial1, W_radial2, W_lin2):
    src = edge_index[0]
    dst = edge_index[1]
    h = _lin1(x, W_lin1)
    radial = _radial(edge_embed.T, W_radial1, W_radial2)
    parts = _sc_scatter(h, src, dst, radial)
    return _final(parts, x, W_lin2)


# speedup vs baseline: 5.4368x; 1.1777x over previous
import functools
import math

import jax
import jax.numpy as jnp
from jax import lax
from jax.experimental import pallas as pl
from jax.experimental.pallas import tpu as pltpu
from jax.experimental.pallas import tpu_sc as plsc

N = 10000
E = 320000
D = 128
NB = 16
H = 64
INV_SQRT_DEG = 1.0 / math.sqrt(32.0)
LOG2 = math.log(2.0)

NC = 2
NS = 16
NW = NC * NS
CHUNK = 64
NCH = 156
EDGES_PER_W = NCH * CHUNK
EX_BASE = NW * EDGES_PER_W
XTRA = (E - EX_BASE) // CHUNK
ROWS_PER_TILE = 624
ROWS_TAIL = N - NS * ROWS_PER_TILE


def _ssp(v):
    return jax.nn.softplus(v) - LOG2


def _lin1_body(x_ref, w_ref, o_ref):
    o_ref[...] = jnp.dot(x_ref[...], w_ref[...], preferred_element_type=jnp.float32)


def _radial_body(eet_ref, w1_ref, w2_ref, o_ref):
    mid = _ssp(lax.dot_general(eet_ref[...], w1_ref[...],
                               (((0,), (0,)), ((), ())),
                               preferred_element_type=jnp.float32))
    o_ref[...] = jnp.dot(mid, w2_ref[...], preferred_element_type=jnp.float32)


def _final_body(parts_ref, x_ref, w_ref, o_ref):
    p = parts_ref[...]
    agg = (p[0] + p[1]) * INV_SQRT_DEG
    o_ref[...] = _ssp(jnp.dot(agg, w_ref[...], preferred_element_type=jnp.float32)) + x_ref[...]


def _lin1(x, w):
    blk = 1000
    return pl.pallas_call(
        _lin1_body,
        grid=(N // blk,),
        in_specs=[pl.BlockSpec((blk, D), lambda i: (i, 0)),
                  pl.BlockSpec((D, D), lambda i: (0, 0))],
        out_specs=pl.BlockSpec((blk, D), lambda i: (i, 0)),
        out_shape=jax.ShapeDtypeStruct((N, D), jnp.float32),
    )(x, w)


def _radial(eet, w1, w2):
    blk = 3200
    return pl.pallas_call(
        _radial_body,
        grid=(E // blk,),
        in_specs=[pl.BlockSpec((NB, blk), lambda i: (0, i)),
                  pl.BlockSpec((NB, H), lambda i: (0, 0)),
                  pl.BlockSpec((H, D), lambda i: (0, 0))],
        out_specs=pl.BlockSpec((blk, D), lambda i: (i, 0)),
        out_shape=jax.ShapeDtypeStruct((E, D), jnp.float32),
    )(eet, w1, w2)


def _final(parts, x, w):
    blk = 1000
    return pl.pallas_call(
        _final_body,
        grid=(N // blk,),
        in_specs=[pl.BlockSpec((NC, blk, D), lambda i: (0, i, 0)),
                  pl.BlockSpec((blk, D), lambda i: (i, 0)),
                  pl.BlockSpec((D, D), lambda i: (0, 0))],
        out_specs=pl.BlockSpec((blk, D), lambda i: (i, 0)),
        out_shape=jax.ShapeDtypeStruct((N, D), jnp.float32),
    )(parts, x, w)


def _sc_body(h_hbm, src_hbm, dst_hbm, radial_hbm, out_hbm,
             src0, src1, src2, dst0, dst1, dst2, dsc0, dsc1, dsc2,
             rows0, rows1, rows2, rad0, rad1, rad2, acc_sh,
             f0, f1, f2, g0, g1, g2, s0, s1, s2):
    cid = lax.axis_index("c")
    sid = lax.axis_index("s")
    wid = sid * NC + cid
    ebase = wid * EDGES_PER_W
    bufs = ((src0, dst0, dsc0, rows0, rad0, f0, g0, s0),
            (src1, dst1, dsc1, rows1, rad1, f1, g1, s1),
            (src2, dst2, dsc2, rows2, rad2, f2, g2, s2))

    def zero_body(c, carry):
        for dd in range(D // 16):
            rows0[c, pl.ds(dd * 16, 16)] = jnp.zeros((16,), jnp.float32)
        return carry
    lax.fori_loop(0, CHUNK, zero_body, 0)

    row0 = sid * ROWS_PER_TILE
    done = 0
    while done < ROWS_PER_TILE:
        n = min(CHUNK, ROWS_PER_TILE - done)
        pltpu.sync_copy(rows0.at[pl.ds(0, n)], acc_sh.at[pl.ds(row0 + done, n)])
        done += n

    @pl.when(sid == NS - 1)
    def _():
        pltpu.sync_copy(rows0.at[pl.ds(0, ROWS_TAIL)],
                        acc_sh.at[pl.ds(NS * ROWS_PER_TILE, ROWS_TAIL)])
    plsc.subcore_barrier()

    def fetch(B, base):
        srcv, dstv, _, _, rad, fsem, _, _ = bufs[B]
        pltpu.async_copy(src_hbm.at[pl.ds(base, CHUNK)], srcv, fsem)
        pltpu.async_copy(dst_hbm.at[pl.ds(base, CHUNK)], dstv, fsem)
        pltpu.async_copy(radial_hbm.at[pl.ds(base, CHUNK)], rad, fsem)

    def wait_fetch(B):
        srcv, dstv, _, _, rad, fsem, _, _ = bufs[B]
        pltpu.make_async_copy(src_hbm.at[pl.ds(0, CHUNK)], srcv, fsem).wait()
        pltpu.make_async_copy(dst_hbm.at[pl.ds(0, CHUNK)], dstv, fsem).wait()
        pltpu.make_async_copy(radial_hbm.at[pl.ds(0, CHUNK)], rad, fsem).wait()

    def gather(B):
        srcv, _, _, rows, _, _, gsem, _ = bufs[B]
        pltpu.async_copy(h_hbm.at[srcv], rows, gsem)

    def wait_gather(B):
        srcv, _, _, rows, _, _, gsem, _ = bufs[B]
        pltpu.make_async_copy(h_hbm.at[srcv], rows, gsem).wait()

    def mult(B):
        _, _, _, rows, rad, _, _, _ = bufs[B]

        def mul_body(c, carry2):
            for dd in range(D // 16):
                sl = pl.ds(dd * 16, 16)
                rows[c, sl] = rows[c, sl] * rad[c, sl]
            return carry2
        lax.fori_loop(0, CHUNK, mul_body, 0)

    def vcopy_dst(B):
        _, dstv, dsc, _, _, _, _, _ = bufs[B]
        for k in range(CHUNK // 16):
            sl = pl.ds(k * 16, 16)
            dsc[sl] = dstv[sl]

    def scat(B):
        _, _, dsc, rows, _, _, _, ssem = bufs[B]
        pltpu.async_copy(rows, acc_sh.at[dsc], ssem, add=True)

    def wait_scat(B):
        _, _, dsc, rows, _, _, _, ssem = bufs[B]
        pltpu.make_async_copy(rows, acc_sh.at[dsc], ssem).wait()

    fetch(0, ebase)
    fetch(1, ebase + CHUNK)
    wait_fetch(0)
    gather(0)

    def phase(j, B):
        nB = (B + 1) % 3
        pB = (B + 2) % 3

        @pl.when(j >= 2)
        def _():
            wait_scat(nB)

        @pl.when(j + 2 < NCH)
        def _():
            fetch(pB, ebase + (j + 2) * CHUNK)

        @pl.when(j + 1 < NCH)
        def _():
            wait_fetch(nB)
            gather(nB)
        wait_gather(B)
        mult(B)
        vcopy_dst(B)
        scat(B)

    def triple_body(it, carry):
        j0 = 3 * it
        phase(j0, 0)
        phase(j0 + 1, 1)
        phase(j0 + 2, 2)
        return carry
    lax.fori_loop(0, NCH // 3, triple_body, 0)

    @pl.when(wid < XTRA)
    def _():
        fetch(0, EX_BASE + wid * CHUNK)
        wait_fetch(0)
        gather(0)
        wait_gather(0)
        mult(0)
        vcopy_dst(0)
        scat(0)
        wait_scat(0)

    wait_scat(1)
    wait_scat(2)

    plsc.subcore_barrier()

    pltpu.sync_copy(acc_sh.at[pl.ds(row0, ROWS_PER_TILE)],
                    out_hbm.at[cid, pl.ds(row0, ROWS_PER_TILE)])

    @pl.when(sid == NS - 1)
    def _():
        pltpu.sync_copy(acc_sh.at[pl.ds(NS * ROWS_PER_TILE, ROWS_TAIL)],
                        out_hbm.at[cid, pl.ds(NS * ROWS_PER_TILE, ROWS_TAIL)])


def _sc_scatter(h, src, dst, radial):
    mesh = plsc.VectorSubcoreMesh(core_axis_name="c", subcore_axis_name="s")
    f = pl.kernel(
        _sc_body, mesh=mesh,
        out_type=jax.ShapeDtypeStruct((NC, N, D), jnp.float32),
        scratch_types=(
            [pltpu.VMEM((CHUNK,), jnp.int32) for _ in range(9)]
            + [pltpu.VMEM((CHUNK, D), jnp.float32) for _ in range(6)]
            + [pltpu.VMEM_SHARED((N, D), jnp.float32)]
            + [pltpu.SemaphoreType.DMA for _ in range(9)]
        ),
    )
    return f(h, src, dst, radial)


def kernel(x, edge_index, edge_embed, W_lin1, W_radial1, W_radial2, W_lin2):
    src = edge_index[0]
    dst = edge_index[1]
    h = _lin1(x, W_lin1)
    radial = _radial(edge_embed.T, W_radial1, W_radial2)
    parts = _sc_scatter(h, src, dst, radial)
    return _final(parts, x, W_lin2)


# speedup vs baseline: 5.9576x; 1.0958x over previous
import functools
import math

import jax
import jax.numpy as jnp
from jax import lax
from jax.experimental import pallas as pl
from jax.experimental.pallas import tpu as pltpu
from jax.experimental.pallas import tpu_sc as plsc

N = 10000
E = 320000
D = 128
NB = 16
H = 64
INV_SQRT_DEG = 1.0 / math.sqrt(32.0)
LOG2 = math.log(2.0)

NC = 2
NS = 16
NW = NC * NS
NSPLIT = 2
ESPLIT = E // NSPLIT
CHUNK = 64
NCH = 78
EDGES_PER_W = NCH * CHUNK
XTRA_REL = NW * EDGES_PER_W
XTRA = (ESPLIT - XTRA_REL) // CHUNK
ROWS_PER_TILE = 624
ROWS_TAIL = N - NS * ROWS_PER_TILE


def _ssp(v):
    return jax.nn.softplus(v) - LOG2


def _lin1_body(x_ref, w_ref, o_ref):
    o_ref[...] = jnp.dot(x_ref[...], w_ref[...], preferred_element_type=jnp.float32)


def _radial_body(eet_ref, w1_ref, w2_ref, o_ref):
    mid = _ssp(lax.dot_general(eet_ref[...], w1_ref[...],
                               (((0,), (0,)), ((), ())),
                               preferred_element_type=jnp.float32))
    o_ref[...] = jnp.dot(mid, w2_ref[...], preferred_element_type=jnp.float32)


def _final_body(parts0_ref, parts1_ref, x_ref, w_ref, o_ref):
    p0 = parts0_ref[...]
    p1 = parts1_ref[...]
    agg = (p0[0] + p0[1] + p1[0] + p1[1]) * INV_SQRT_DEG
    o_ref[...] = _ssp(jnp.dot(agg, w_ref[...], preferred_element_type=jnp.float32)) + x_ref[...]


def _lin1(x, w):
    blk = 1000
    return pl.pallas_call(
        _lin1_body,
        grid=(N // blk,),
        in_specs=[pl.BlockSpec((blk, D), lambda i: (i, 0)),
                  pl.BlockSpec((D, D), lambda i: (0, 0))],
        out_specs=pl.BlockSpec((blk, D), lambda i: (i, 0)),
        out_shape=jax.ShapeDtypeStruct((N, D), jnp.float32),
    )(x, w)


def _radial(eet, w1, w2, koff):
    blk = 3200
    return pl.pallas_call(
        _radial_body,
        grid=(ESPLIT // blk,),
        in_specs=[pl.BlockSpec((NB, blk), lambda i, koff=koff: (0, i + koff)),
                  pl.BlockSpec((NB, H), lambda i: (0, 0)),
                  pl.BlockSpec((H, D), lambda i: (0, 0))],
        out_specs=pl.BlockSpec((blk, D), lambda i: (i, 0)),
        out_shape=jax.ShapeDtypeStruct((ESPLIT, D), jnp.float32),
    )(eet, w1, w2)


def _final(parts0, parts1, x, w):
    blk = 1000
    return pl.pallas_call(
        _final_body,
        grid=(N // blk,),
        in_specs=[pl.BlockSpec((NC, blk, D), lambda i: (0, i, 0)),
                  pl.BlockSpec((NC, blk, D), lambda i: (0, i, 0)),
                  pl.BlockSpec((blk, D), lambda i: (i, 0)),
                  pl.BlockSpec((D, D), lambda i: (0, 0))],
        out_specs=pl.BlockSpec((blk, D), lambda i: (i, 0)),
        out_shape=jax.ShapeDtypeStruct((N, D), jnp.float32),
    )(parts0, parts1, x, w)


def _when(pred, fn):
    if isinstance(pred, bool):
        if pred:
            fn()
    else:
        pl.when(pred)(fn)


def _sc_body(lo,
             h_hbm, src_hbm, dst_hbm, radial_hbm, out_hbm,
             src0, src1, src2, dst0, dst1, dst2, dsc0, dsc1, dsc2,
             rows0, rows1, rows2, rad0, rad1, rad2, acc_sh,
             f0, f1, f2, g0, g1, g2, s0, s1, s2):
    cid = lax.axis_index("c")
    sid = lax.axis_index("s")
    wid = sid * NC + cid
    rbase = wid * EDGES_PER_W
    ebase = lo + rbase
    bufs = ((src0, dst0, dsc0, rows0, rad0, f0, g0, s0),
            (src1, dst1, dsc1, rows1, rad1, f1, g1, s1),
            (src2, dst2, dsc2, rows2, rad2, f2, g2, s2))

    def zero_body(c, carry):
        for dd in range(D // 16):
            rows0[c, pl.ds(dd * 16, 16)] = jnp.zeros((16,), jnp.float32)
        return carry
    lax.fori_loop(0, CHUNK, zero_body, 0)

    row0 = sid * ROWS_PER_TILE
    done = 0
    while done < ROWS_PER_TILE:
        n = min(CHUNK, ROWS_PER_TILE - done)
        pltpu.sync_copy(rows0.at[pl.ds(0, n)], acc_sh.at[pl.ds(row0 + done, n)])
        done += n

    @pl.when(sid == NS - 1)
    def _():
        pltpu.sync_copy(rows0.at[pl.ds(0, ROWS_TAIL)],
                        acc_sh.at[pl.ds(NS * ROWS_PER_TILE, ROWS_TAIL)])
    plsc.subcore_barrier()

    def fetch(B, j):
        srcv, dstv, _, _, rad, fsem, _, _ = bufs[B]
        pltpu.async_copy(src_hbm.at[pl.ds(ebase + j * CHUNK, CHUNK)], srcv, fsem)
        pltpu.async_copy(dst_hbm.at[pl.ds(ebase + j * CHUNK, CHUNK)], dstv, fsem)
        pltpu.async_copy(radial_hbm.at[pl.ds(rbase + j * CHUNK, CHUNK)], rad, fsem)

    def wait_fetch(B):
        srcv, dstv, _, _, rad, fsem, _, _ = bufs[B]
        pltpu.make_async_copy(src_hbm.at[pl.ds(0, CHUNK)], srcv, fsem).wait()
        pltpu.make_async_copy(dst_hbm.at[pl.ds(0, CHUNK)], dstv, fsem).wait()
        pltpu.make_async_copy(radial_hbm.at[pl.ds(0, CHUNK)], rad, fsem).wait()

    def gather(B):
        srcv, _, _, rows, _, _, gsem, _ = bufs[B]
        pltpu.async_copy(h_hbm.at[srcv], rows, gsem)

    def wait_gather(B):
        srcv, _, _, rows, _, _, gsem, _ = bufs[B]
        pltpu.make_async_copy(h_hbm.at[srcv], rows, gsem).wait()

    def mult(B):
        _, _, _, rows, rad, _, _, _ = bufs[B]

        def mul_body(c, carry2):
            for dd in range(D // 16):
                sl = pl.ds(dd * 16, 16)
                rows[c, sl] = rows[c, sl] * rad[c, sl]
            return carry2
        lax.fori_loop(0, CHUNK, mul_body, 0)

    def vcopy_dst(B):
        _, dstv, dsc, _, _, _, _, _ = bufs[B]
        for k in range(CHUNK // 16):
            sl = pl.ds(k * 16, 16)
            dsc[sl] = dstv[sl]

    def scat(B):
        _, _, dsc, rows, _, _, _, ssem = bufs[B]
        pltpu.async_copy(rows, acc_sh.at[dsc], ssem, add=True)

    def wait_scat(B):
        _, _, dsc, rows, _, _, _, ssem = bufs[B]
        pltpu.make_async_copy(rows, acc_sh.at[dsc], ssem).wait()

    fetch(0, 0)
    fetch(1, 1)
    wait_fetch(0)
    gather(0)

    def phase(j, B):
        nB = (B + 1) % 3
        pB = (B + 2) % 3
        _when(j >= 2, lambda: wait_scat(nB))
        _when(j + 2 < NCH, lambda: fetch(pB, j + 2))

        def _adv():
            wait_fetch(nB)
            gather(nB)
        _when(j + 1 < NCH, _adv)
        wait_gather(B)
        mult(B)
        vcopy_dst(B)
        scat(B)

    def triple_body(it, carry):
        j0 = 3 * it
        phase(j0, 0)
        phase(j0 + 1, 1)
        phase(j0 + 2, 2)
        return carry
    lax.fori_loop(0, NCH // 3, triple_body, 0)

    @pl.when(wid < XTRA)
    def _():
        srcv, dstv, _, rows, rad, fsem, _, _ = bufs[0]
        pltpu.async_copy(src_hbm.at[pl.ds(lo + XTRA_REL + wid * CHUNK, CHUNK)],
                         srcv, fsem)
        pltpu.async_copy(dst_hbm.at[pl.ds(lo + XTRA_REL + wid * CHUNK, CHUNK)],
                         dstv, fsem)
        pltpu.async_copy(radial_hbm.at[pl.ds(XTRA_REL + wid * CHUNK, CHUNK)],
                         rad, fsem)
        wait_fetch(0)
        gather(0)
        wait_gather(0)
        mult(0)
        vcopy_dst(0)
        scat(0)
        wait_scat(0)

    wait_scat(1)
    wait_scat(2)

    plsc.subcore_barrier()

    pltpu.sync_copy(acc_sh.at[pl.ds(row0, ROWS_PER_TILE)],
                    out_hbm.at[cid, pl.ds(row0, ROWS_PER_TILE)])

    @pl.when(sid == NS - 1)
    def _():
        pltpu.sync_copy(acc_sh.at[pl.ds(NS * ROWS_PER_TILE, ROWS_TAIL)],
                        out_hbm.at[cid, pl.ds(NS * ROWS_PER_TILE, ROWS_TAIL)])


def _sc_scatter(h, src, dst, radial_k, lo):
    mesh = plsc.VectorSubcoreMesh(core_axis_name="c", subcore_axis_name="s")
    f = pl.kernel(
        functools.partial(_sc_body, lo), mesh=mesh,
        out_type=jax.ShapeDtypeStruct((NC, N, D), jnp.float32),
        scratch_types=(
            [pltpu.VMEM((CHUNK,), jnp.int32) for _ in range(9)]
            + [pltpu.VMEM((CHUNK, D), jnp.float32) for _ in range(6)]
            + [pltpu.VMEM_SHARED((N, D), jnp.float32)]
            + [pltpu.SemaphoreType.DMA for _ in range(9)]
        ),
    )
    return f(h, src, dst, radial_k)


def kernel(x, edge_index, edge_embed, W_lin1, W_radial1, W_radial2, W_lin2):
    src = edge_index[0]
    dst = edge_index[1]
    eet = edge_embed.T
    h = _lin1(x, W_lin1)
    radial0 = _radial(eet, W_radial1, W_radial2, 0)
    parts0 = _sc_scatter(h, src, dst, radial0, 0)
    radial1 = _radial(eet, W_radial1, W_radial2, ESPLIT // 3200)
    h1, src1, dst1, radial1b, _ = jax.lax.optimization_barrier(
        (h, src, dst, radial1, parts0))
    parts1 = _sc_scatter(h1, src1, dst1, radial1b, ESPLIT)
    return _final(parts0, parts1, x, W_lin2)


# speedup vs baseline: 6.2219x; 1.0444x over previous
import functools
import math

import jax
import jax.numpy as jnp
from jax import lax
from jax.experimental import pallas as pl
from jax.experimental.pallas import tpu as pltpu
from jax.experimental.pallas import tpu_sc as plsc

N = 10000
E = 320000
D = 128
NB = 16
H = 64
INV_SQRT_DEG = 1.0 / math.sqrt(32.0)
LOG2 = math.log(2.0)

NC = 2
NS = 16
NW = NC * NS
NSPLIT = 2
ESPLIT = E // NSPLIT
CHUNK = 64
NCH = 78
EDGES_PER_W = NCH * CHUNK
XTRA_REL = NW * EDGES_PER_W
XTRA = (ESPLIT - XTRA_REL) // CHUNK
ROWS_PER_TILE = 624
ROWS_TAIL = N - NS * ROWS_PER_TILE


def _ssp(v):
    return jax.nn.softplus(v) - LOG2


def _lin1_body(x_ref, w_ref, o_ref):
    o_ref[...] = jnp.dot(x_ref[...], w_ref[...], preferred_element_type=jnp.float32)


def _radial_body(eet_ref, w1_ref, w2_ref, o_ref):
    mid = _ssp(lax.dot_general(eet_ref[...], w1_ref[...],
                               (((0,), (0,)), ((), ())),
                               preferred_element_type=jnp.float32))
    o_ref[...] = jnp.dot(mid, w2_ref[...], preferred_element_type=jnp.float32)


def _final_body(parts0_ref, parts1_ref, x_ref, w_ref, o_ref):
    p0 = parts0_ref[...]
    p1 = parts1_ref[...]
    agg = (p0[0] + p0[1] + p1[0] + p1[1]) * INV_SQRT_DEG
    o_ref[...] = _ssp(jnp.dot(agg, w_ref[...], preferred_element_type=jnp.float32)) + x_ref[...]


def _lin1(x, w):
    blk = 1000
    return pl.pallas_call(
        _lin1_body,
        grid=(N // blk,),
        in_specs=[pl.BlockSpec((blk, D), lambda i: (i, 0)),
                  pl.BlockSpec((D, D), lambda i: (0, 0))],
        out_specs=pl.BlockSpec((blk, D), lambda i: (i, 0)),
        out_shape=jax.ShapeDtypeStruct((N, D), jnp.float32),
    )(x, w)


def _radial(eet, w1, w2, koff):
    blk = 3200
    return pl.pallas_call(
        _radial_body,
        grid=(ESPLIT // blk,),
        in_specs=[pl.BlockSpec((NB, blk), lambda i, koff=koff: (0, i + koff)),
                  pl.BlockSpec((NB, H), lambda i: (0, 0)),
                  pl.BlockSpec((H, D), lambda i: (0, 0))],
        out_specs=pl.BlockSpec((blk, D), lambda i: (i, 0)),
        out_shape=jax.ShapeDtypeStruct((ESPLIT, D), jnp.float32),
    )(eet, w1, w2)


def _final(parts0, parts1, x, w):
    blk = 1000
    return pl.pallas_call(
        _final_body,
        grid=(N // blk,),
        in_specs=[pl.BlockSpec((NC, blk, D), lambda i: (0, i, 0)),
                  pl.BlockSpec((NC, blk, D), lambda i: (0, i, 0)),
                  pl.BlockSpec((blk, D), lambda i: (i, 0)),
                  pl.BlockSpec((D, D), lambda i: (0, 0))],
        out_specs=pl.BlockSpec((blk, D), lambda i: (i, 0)),
        out_shape=jax.ShapeDtypeStruct((N, D), jnp.float32),
    )(parts0, parts1, x, w)


def _when(pred, fn):
    if isinstance(pred, bool):
        if pred:
            fn()
    else:
        pl.when(pred)(fn)


def _sc_body(lo,
             h_hbm, src_hbm, dst_hbm, radial_hbm, out_hbm,
             src0, src1, src2, dst0, dst1, dst2, dsc0, dsc1, dsc2,
             rows0, rows1, rows2, rad0, rad1, rad2, acc_sh,
             f0, f1, f2, g0, g1, g2, s0, s1, s2):
    cid = lax.axis_index("c")
    sid = lax.axis_index("s")
    wid = sid * NC + cid
    rbase = wid * EDGES_PER_W
    ebase = lo + rbase
    bufs = ((src0, dst0, dsc0, rows0, rad0, f0, g0, s0),
            (src1, dst1, dsc1, rows1, rad1, f1, g1, s1),
            (src2, dst2, dsc2, rows2, rad2, f2, g2, s2))

    def zero_body(c, carry):
        for dd in range(D // 16):
            rows0[c, pl.ds(dd * 16, 16)] = jnp.zeros((16,), jnp.float32)
        return carry
    lax.fori_loop(0, CHUNK, zero_body, 0)

    row0 = sid * ROWS_PER_TILE
    done = 0
    while done < ROWS_PER_TILE:
        n = min(CHUNK, ROWS_PER_TILE - done)
        pltpu.sync_copy(rows0.at[pl.ds(0, n)], acc_sh.at[pl.ds(row0 + done, n)])
        done += n

    @pl.when(sid == NS - 1)
    def _():
        pltpu.sync_copy(rows0.at[pl.ds(0, ROWS_TAIL)],
                        acc_sh.at[pl.ds(NS * ROWS_PER_TILE, ROWS_TAIL)])
    plsc.subcore_barrier()

    def fetch(B, j):
        srcv, dstv, _, _, rad, fsem, _, _ = bufs[B]
        pltpu.async_copy(src_hbm.at[pl.ds(ebase + j * CHUNK, CHUNK)], srcv, fsem)
        pltpu.async_copy(dst_hbm.at[pl.ds(ebase + j * CHUNK, CHUNK)], dstv, fsem)
        pltpu.async_copy(radial_hbm.at[pl.ds(rbase + j * CHUNK, CHUNK)], rad, fsem)

    def wait_fetch(B):
        srcv, dstv, _, _, rad, fsem, _, _ = bufs[B]
        pltpu.make_async_copy(src_hbm.at[pl.ds(0, CHUNK)], srcv, fsem).wait()
        pltpu.make_async_copy(dst_hbm.at[pl.ds(0, CHUNK)], dstv, fsem).wait()
        pltpu.make_async_copy(radial_hbm.at[pl.ds(0, CHUNK)], rad, fsem).wait()

    def gather(B):
        srcv, _, _, rows, _, _, gsem, _ = bufs[B]
        pltpu.async_copy(h_hbm.at[srcv], rows, gsem)

    def wait_gather(B):
        srcv, _, _, rows, _, _, gsem, _ = bufs[B]
        pltpu.make_async_copy(h_hbm.at[srcv], rows, gsem).wait()

    def mult(B):
        _, _, _, rows, rad, _, _, _ = bufs[B]

        @functools.partial(plsc.parallel_loop, 0, CHUNK, unroll=4)
        def _(c):
            for dd in range(D // 16):
                sl = pl.ds(dd * 16, 16)
                rows[c, sl] = rows[c, sl] * rad[c, sl]

    def vcopy_dst(B):
        _, dstv, dsc, _, _, _, _, _ = bufs[B]
        for k in range(CHUNK // 16):
            sl = pl.ds(k * 16, 16)
            dsc[sl] = dstv[sl]

    def scat(B):
        _, _, dsc, rows, _, _, _, ssem = bufs[B]
        pltpu.async_copy(rows, acc_sh.at[dsc], ssem, add=True)

    def wait_scat(B):
        _, _, dsc, rows, _, _, _, ssem = bufs[B]
        pltpu.make_async_copy(rows, acc_sh.at[dsc], ssem).wait()

    fetch(0, 0)
    fetch(1, 1)
    wait_fetch(0)
    gather(0)

    def phase(j, B):
        nB = (B + 1) % 3
        pB = (B + 2) % 3
        _when(j >= 2, lambda: wait_scat(nB))
        _when(j + 2 < NCH, lambda: fetch(pB, j + 2))

        def _adv():
            wait_fetch(nB)
            gather(nB)
        _when(j + 1 < NCH, _adv)
        wait_gather(B)
        mult(B)
        vcopy_dst(B)
        scat(B)

    def triple_body(it, carry):
        j0 = 3 * it
        phase(j0, 0)
        phase(j0 + 1, 1)
        phase(j0 + 2, 2)
        return carry
    lax.fori_loop(0, NCH // 3, triple_body, 0)

    @pl.when(wid < XTRA)
    def _():
        srcv, dstv, _, rows, rad, fsem, _, _ = bufs[0]
        pltpu.async_copy(src_hbm.at[pl.ds(lo + XTRA_REL + wid * CHUNK, CHUNK)],
                         srcv, fsem)
        pltpu.async_copy(dst_hbm.at[pl.ds(lo + XTRA_REL + wid * CHUNK, CHUNK)],
                         dstv, fsem)
        pltpu.async_copy(radial_hbm.at[pl.ds(XTRA_REL + wid * CHUNK, CHUNK)],
                         rad, fsem)
        wait_fetch(0)
        gather(0)
        wait_gather(0)
        mult(0)
        vcopy_dst(0)
        scat(0)
        wait_scat(0)

    wait_scat(1)
    wait_scat(2)

    plsc.subcore_barrier()

    pltpu.sync_copy(acc_sh.at[pl.ds(row0, ROWS_PER_TILE)],
                    out_hbm.at[cid, pl.ds(row0, ROWS_PER_TILE)])

    @pl.when(sid == NS - 1)
    def _():
        pltpu.sync_copy(acc_sh.at[pl.ds(NS * ROWS_PER_TILE, ROWS_TAIL)],
                        out_hbm.at[cid, pl.ds(NS * ROWS_PER_TILE, ROWS_TAIL)])


def _sc_scatter(h, src, dst, radial_k, lo):
    mesh = plsc.VectorSubcoreMesh(core_axis_name="c", subcore_axis_name="s")
    f = pl.kernel(
        functools.partial(_sc_body, lo), mesh=mesh,
        out_type=jax.ShapeDtypeStruct((NC, N, D), jnp.float32),
        scratch_types=(
            [pltpu.VMEM((CHUNK,), jnp.int32) for _ in range(9)]
            + [pltpu.VMEM((CHUNK, D), jnp.float32) for _ in range(6)]
            + [pltpu.VMEM_SHARED((N, D), jnp.float32)]
            + [pltpu.SemaphoreType.DMA for _ in range(9)]
        ),
    )
    return f(h, src, dst, radial_k)


def kernel(x, edge_index, edge_embed, W_lin1, W_radial1, W_radial2, W_lin2):
    src = edge_index[0]
    dst = edge_index[1]
    eet = edge_embed.T
    h = _lin1(x, W_lin1)
    radial0 = _radial(eet, W_radial1, W_radial2, 0)
    parts0 = _sc_scatter(h, src, dst, radial0, 0)
    radial1 = _radial(eet, W_radial1, W_radial2, ESPLIT // 3200)
    h1, src1, dst1, radial1b, _ = jax.lax.optimization_barrier(
        (h, src, dst, radial1, parts0))
    parts1 = _sc_scatter(h1, src1, dst1, radial1b, ESPLIT)
    return _final(parts0, parts1, x, W_lin2)
